# baseline (speedup 1.0000x reference)
import functools
import math

import jax
import jax.numpy as jnp
from jax import lax
from jax.experimental import pallas as pl
from jax.experimental.pallas import tpu as pltpu
from jax.experimental.pallas import tpu_sc as plsc

N = 48758
E = 780128
NP = 49152
EP = 786432
C = 2048
HID = 64
BLK = 512
GRID = NP // BLK
F32 = jnp.float32
I32 = jnp.int32

_mesh = plsc.VectorSubcoreMesh(
    core_axis_name="c", subcore_axis_name="s", num_cores=2, num_subcores=16
)
_sc_params = pltpu.CompilerParams(
    needs_layout_passes=False, use_tc_tiling_on_sc=False
)


def _k1_body(x_ref, w_ref, b_ref, h_ref, s_ref, q_ref):
    i = pl.program_id(0)
    h = jnp.dot(x_ref[...], w_ref[...], preferred_element_type=F32) + b_ref[...]
    h_ref[...] = h
    rid = i * BLK + lax.broadcasted_iota(I32, (BLK, 1), 0)
    hm = jnp.where(rid < N, h, 0.0)

    @pl.when(i == 0)
    def _():
        s_ref[...] = jnp.zeros_like(s_ref)
        q_ref[...] = jnp.zeros_like(q_ref)

    s_ref[0:1, :] += jnp.sum(hm, axis=0, keepdims=True)
    q_ref[0:1, :] += jnp.sum(hm * hm, axis=0, keepdims=True)


_k1 = pl.pallas_call(
    _k1_body,
    grid=(GRID,),
    in_specs=[
        pl.BlockSpec((BLK, 128), lambda i: (i, 0)),
        pl.BlockSpec((128, HID), lambda i: (0, 0)),
        pl.BlockSpec((1, HID), lambda i: (0, 0)),
    ],
    out_specs=[
        pl.BlockSpec((BLK, HID), lambda i: (i, 0)),
        pl.BlockSpec((8, HID), lambda i: (0, 0)),
        pl.BlockSpec((8, HID), lambda i: (0, 0)),
    ],
    out_shape=[
        jax.ShapeDtypeStruct((NP, HID), F32),
        jax.ShapeDtypeStruct((8, HID), F32),
        jax.ShapeDtypeStruct((8, HID), F32),
    ],
)


def _k2_body(h_ref, s_ref, q_ref, g_ref, bb_ref, wi_ref, bi_ref, wrel_ref,
             degp_ref, xf_ref, dis_ref):
    mean = s_ref[0:1, :] / N
    var = q_ref[0:1, :] / N - mean * mean
    rstd = lax.rsqrt(var + 1e-5)
    cc = jnp.maximum((h_ref[...] - mean) * rstd * g_ref[...] + bb_ref[...], 0.0)
    xin = jnp.dot(cc, wi_ref[...], preferred_element_type=F32) + bi_ref[...]
    deg = degp_ref[0] + degp_ref[1]
    dis = jnp.where(deg > 0, lax.rsqrt(deg), 0.0)
    dis_ref[...] = dis
    xp = xin * dis
    for r in range(4):
        xf_ref[r, :, 0:HID] = xp
        xf_ref[r, :, HID:2 * HID] = jnp.dot(
            xin, wrel_ref[r], preferred_element_type=F32)


_k2 = pl.pallas_call(
    _k2_body,
    grid=(GRID,),
    in_specs=[
        pl.BlockSpec((BLK, HID), lambda i: (i, 0)),
        pl.BlockSpec((8, HID), lambda i: (0, 0)),
        pl.BlockSpec((8, HID), lambda i: (0, 0)),
        pl.BlockSpec((1, HID), lambda i: (0, 0)),
        pl.BlockSpec((1, HID), lambda i: (0, 0)),
        pl.BlockSpec((HID, HID), lambda i: (0, 0)),
        pl.BlockSpec((1, HID), lambda i: (0, 0)),
        pl.BlockSpec((4, HID, HID), lambda i: (0, 0, 0)),
        pl.BlockSpec((2, BLK, 1), lambda i: (0, i, 0)),
    ],
    out_specs=[
        pl.BlockSpec((4, BLK, 2 * HID), lambda i: (0, i, 0)),
        pl.BlockSpec((BLK, 1), lambda i: (i, 0)),
    ],
    out_shape=[
        jax.ShapeDtypeStruct((4, NP, 2 * HID), F32),
        jax.ShapeDtypeStruct((NP, 1), F32),
    ],
)


def _k3_body(cb_ref, wo_ref, bo_ref, wi_ref, bi_ref, wrel_ref, dis_ref,
             xf_ref):
    c1 = jnp.dot(cb_ref[...], wo_ref[...], preferred_element_type=F32) + bo_ref[...]
    xin = jnp.dot(c1, wi_ref[...], preferred_element_type=F32) + bi_ref[...]
    xp = xin * dis_ref[...]
    for r in range(4):
        xf_ref[r, :, 0:HID] = xp
        xf_ref[r, :, HID:2 * HID] = jnp.dot(
            xin, wrel_ref[r], preferred_element_type=F32)


_k3 = pl.pallas_call(
    _k3_body,
    grid=(GRID,),
    in_specs=[
        pl.BlockSpec((BLK, HID), lambda i: (i, 0)),
        pl.BlockSpec((HID, HID), lambda i: (0, 0)),
        pl.BlockSpec((1, HID), lambda i: (0, 0)),
        pl.BlockSpec((HID, HID), lambda i: (0, 0)),
        pl.BlockSpec((1, HID), lambda i: (0, 0)),
        pl.BlockSpec((4, HID, HID), lambda i: (0, 0, 0)),
        pl.BlockSpec((BLK, 1), lambda i: (i, 0)),
    ],
    out_specs=[
        pl.BlockSpec((4, BLK, 2 * HID), lambda i: (0, i, 0)),
    ],
    out_shape=[
        jax.ShapeDtypeStruct((4, NP, 2 * HID), F32),
    ],
)


def _k5_body(rows_ref, wo_ref, bo_ref, out_ref):
    o = jnp.dot(rows_ref[...], wo_ref[...], preferred_element_type=F32) + bo_ref[...]
    out_ref[...] = 0.5 * o * (1.0 + lax.erf(o * (1.0 / math.sqrt(2.0))))


_k5 = pl.pallas_call(
    _k5_body,
    grid=(16,),
    in_specs=[
        pl.BlockSpec((BLK, HID), lambda i: (i, 0)),
        pl.BlockSpec((HID, HID), lambda i: (0, 0)),
        pl.BlockSpec((1, HID), lambda i: (0, 0)),
    ],
    out_specs=pl.BlockSpec((BLK, HID), lambda i: (i, 0)),
    out_shape=jax.ShapeDtypeStruct((8192, HID), F32),
)


@functools.partial(
    pl.kernel,
    mesh=_mesh,
    out_type=jax.ShapeDtypeStruct((2, NP), F32),
    scratch_types=[
        pltpu.VMEM((2048,), I32),
        pltpu.VMEM((128,), I32),
        pltpu.VMEM((128,), F32),
        pltpu.VMEM((3072,), F32),
        pltpu.VMEM_SHARED((NP,), F32),
    ],
    compiler_params=_sc_params,
)
def _deg_kernel(col_hbm, out_hbm, colstage, idxbuf, ones_v, zbuf, acc):
    cid = lax.axis_index("c")
    sid = lax.axis_index("s")
    wid = sid * 2 + cid

    def initz(i, _):
        zbuf[pl.ds(i * 16, 16)] = jnp.zeros((16,), F32)
        return 0

    lax.fori_loop(0, 192, initz, 0)
    for j in range(8):
        ones_v[pl.ds(j * 16, 16)] = jnp.ones((16,), F32)
    pltpu.sync_copy(zbuf, acc.at[pl.ds(sid * 3072, 3072)])
    plsc.subcore_barrier()

    base = wid * (EP // 32)

    def chunk_body(ch, _):
        pltpu.sync_copy(col_hbm.at[pl.ds(base + ch * 2048, 2048)], colstage)

        def batch_body(b, _2):
            for j in range(8):
                idxbuf[pl.ds(j * 16, 16)] = colstage[pl.ds(b * 128 + j * 16, 16)]
            pltpu.sync_copy(ones_v, acc.at[idxbuf], add=True)
            return 0

        lax.fori_loop(0, 16, batch_body, 0)
        return 0

    lax.fori_loop(0, (EP // 32) // 2048, chunk_body, 0)
    plsc.subcore_barrier()
    pltpu.sync_copy(
        acc.at[pl.ds(sid * 3072, 3072)], out_hbm.at[cid, pl.ds(sid * 3072, 3072)]
    )


_STAG = 2304


@functools.partial(
    pl.kernel,
    mesh=_mesh,
    out_type=jax.ShapeDtypeStruct((NP, HID), F32),
    scratch_types=[
        pltpu.VMEM((2048,), I32),
        pltpu.VMEM((2048,), I32),
        pltpu.VMEM((_STAG,), I32),
        pltpu.VMEM((_STAG,), I32),
        pltpu.VMEM((128,), I32),
        pltpu.VMEM((128,), I32),
        pltpu.VMEM((128, 2 * HID), F32),
        pltpu.VMEM((128, 3 * HID), F32),
        pltpu.VMEM((128, 3 * HID), F32),
        pltpu.VMEM((128, HID), F32),
        pltpu.VMEM((128,), F32),
        pltpu.VMEM_SHARED((C + 16, 3 * HID), F32),
        pltpu.SemaphoreType.DMA,
    ],
    compiler_params=_sc_params,
)
def _edge_kernel(row_hbm, ct_hbm, xf_hbm, dis_hbm, out_hbm,
                 rstage, ctstage, rls, xs, rlbuf, xbuf,
                 Frows, U, zbuf, ob, disloc, acc, semx):
    cid = lax.axis_index("c")
    sid = lax.axis_index("s")

    def initzb(e, _):
        zr = zbuf.at[e]
        for cch in range(12):
            zr[pl.ds(cch * 16, 16)] = jnp.zeros((16,), F32)
        return 0

    lax.fori_loop(0, 128, initzb, 0)

    def inits(i, _):
        fill = jnp.full((16,), 0, I32) + sid
        xs[pl.ds(i * 16, 16)] = fill
        rls[pl.ds(i * 16, 16)] = jnp.full((16,), C, I32)
        return 0

    lax.fori_loop(0, _STAG // 16, inits, 0)

    def process_batch(o):
        for j in range(8):
            rlbuf[pl.ds(j * 16, 16)] = rls[pl.ds(o + j * 16, 16)]
            xbuf[pl.ds(j * 16, 16)] = xs[pl.ds(o + j * 16, 16)]
        pltpu.async_copy(xf_hbm.at[xbuf], Frows, semx).wait()

        def ubody(e, _):
            fr = Frows.at[e]
            ur = U.at[e]
            for cch in range(4):
                ur[pl.ds(cch * 16, 16)] = fr[pl.ds(cch * 16, 16)]
            for cch in range(4):
                v = fr[pl.ds(HID + cch * 16, 16)]
                p = jnp.exp(v)
                ur[pl.ds(HID + cch * 16, 16)] = p
                ur[pl.ds(2 * HID + cch * 16, 16)] = v * p
            return 0

        lax.fori_loop(0, 128, ubody, 0)
        pltpu.sync_copy(U, acc.at[rlbuf], add=True)

    def pass_body(rid_pass, _pb):
        rid = rid_pass * 2 + cid
        base = rid * C

        r0 = sid * 129
        pltpu.sync_copy(zbuf, acc.at[pl.ds(r0, 128)])
        pltpu.sync_copy(zbuf.at[pl.ds(0, 1)], acc.at[pl.ds(r0 + 128, 1)])
        plsc.subcore_barrier()

        def chunk_body(ch, n_in):
            eb = sid * (EP // 16) + ch * 2048
            pltpu.sync_copy(row_hbm.at[pl.ds(eb, 2048)], rstage)
            pltpu.sync_copy(ct_hbm.at[pl.ds(eb, 2048)], ctstage)

            def grp(g, n):
                r16 = rstage[pl.ds(g * 16, 16)]
                ct16 = ctstage[pl.ds(g * 16, 16)]
                m = (r16 >= base) & (r16 < base + C)
                mi = jnp.where(m, 1, 0)
                cs_ = plsc.cumsum(mi)
                pos = n + cs_ - 1
                gi = lax.shift_right_logical(ct16, 16) * NP + (ct16 & 0xFFFF)
                plsc.store_scatter(rls, [pos], r16 - base, mask=m)
                plsc.store_scatter(xs, [pos], gi, mask=m)
                return n + cs_[15]

            n = lax.fori_loop(0, 128, grp, n_in)

            def dcond(st):
                return st[0] + 128 <= st[1]

            def dbody(st):
                nd, ns = st
                process_batch(nd)
                return (nd + 128, ns)

            nd, ns = lax.while_loop(dcond, dbody, (0, n))

            @pl.when(nd > 0)
            def _():
                for j in range(8):
                    rls[pl.ds(j * 16, 16)] = rls[pl.ds(nd + j * 16, 16)]
                    xs[pl.ds(j * 16, 16)] = xs[pl.ds(nd + j * 16, 16)]

            return ns - nd

        n_left = lax.fori_loop(0, (EP // 16) // 2048, chunk_body, 0)

        @pl.when(n_left > 0)
        def _():
            iota = jnp.arange(16, dtype=I32)
            dump = jnp.full((16,), C, I32)
            for j in range(8):
                plsc.store_scatter(rls, [n_left + j * 16 + iota], dump)
            process_batch(0)

        plsc.subcore_barrier()

        gbase = base + sid * 128
        pltpu.sync_copy(dis_hbm.at[pl.ds(gbase, 128)], disloc)
        pltpu.sync_copy(acc.at[pl.ds(sid * 128, 128)], U)

        def pgroup(g, _):
            dv16 = disloc[pl.ds(g * 16, 16)]
            for k in range(16):
                i = g * 16 + k
                dv = dv16[k]
                ur = U.at[i]
                ro = ob.at[i]
                for cch in range(4):
                    a = ur[pl.ds(cch * 16, 16)]
                    s = ur[pl.ds(HID + cch * 16, 16)]
                    w = ur[pl.ds(2 * HID + cch * 16, 16)]
                    msg = w / (s + 1e-16)
                    ro[pl.ds(cch * 16, 16)] = dv * a + 0.1 * jnp.maximum(msg, 0.0)
            return 0

        lax.fori_loop(0, 8, pgroup, 0)
        pltpu.sync_copy(ob, out_hbm.at[pl.ds(gbase, 128)])
        plsc.subcore_barrier()
        return 0

    lax.fori_loop(0, 12, pass_body, 0)


@functools.partial(
    pl.kernel,
    mesh=_mesh,
    out_type=jax.ShapeDtypeStruct((8192, HID), F32),
    scratch_types=[
        pltpu.VMEM((128,), I32),
        pltpu.VMEM((128, HID), F32),
        pltpu.SemaphoreType.DMA,
    ],
    compiler_params=_sc_params,
)
def _gather_kernel(src_hbm, idx_hbm, out_hbm, ib, rows, sem):
    cid = lax.axis_index("c")
    sid = lax.axis_index("s")
    wid = sid * 2 + cid
    for b in range(2):
        o = wid * 256 + b * 128
        pltpu.sync_copy(idx_hbm.at[pl.ds(o, 128)], ib)
        pltpu.async_copy(src_hbm.at[ib], rows, sem).wait()
        pltpu.sync_copy(rows, out_hbm.at[pl.ds(o, 128)])


def kernel(x, edge_index, idx, edge_type, edge_weight, proj_W, proj_b, bn_g,
           bn_b, Wi0, bi0, Wrel0, Wo0, bo0, Wi1, bi1, Wrel1, Wo1, bo1):
    del edge_weight
    x_p = jnp.pad(x, ((0, NP - N), (0, 0)))
    row = edge_index[0]
    col = edge_index[1]
    pad_e = EP - E
    ar = jnp.arange(pad_e, dtype=I32)
    row_p = jnp.concatenate([row, jnp.full((pad_e,), 1 << 20, I32)])
    col_p = jnp.concatenate([col, N + ar % (NP - N)])
    typ_p = jnp.concatenate([edge_type, jnp.zeros((pad_e,), I32)])
    ct_p = (typ_p << 16) | col_p

    degp = _deg_kernel(col_p)
    h, hsum, hsq = _k1(x_p, proj_W, proj_b.reshape(1, HID))
    xf0, dis = _k2(
        h, hsum, hsq, bn_g.reshape(1, HID), bn_b.reshape(1, HID), Wi0,
        bi0.reshape(1, HID), Wrel0, degp.reshape(2, NP, 1))
    dis_flat = dis.reshape(NP)
    comb0 = _edge_kernel(row_p, ct_p, xf0.reshape(4 * NP, 2 * HID), dis_flat)
    (xf1,) = _k3(comb0, Wo0, bo0.reshape(1, HID), Wi1,
                 bi1.reshape(1, HID), Wrel1, dis)
    comb1 = _edge_kernel(row_p, ct_p, xf1.reshape(4 * NP, 2 * HID), dis_flat)
    rows = _gather_kernel(comb1, idx)
    return _k5(rows, Wo1, bo1.reshape(1, HID))


# speedup vs baseline: 1.0092x; 1.0092x over previous
import functools
import math

import jax
import jax.numpy as jnp
from jax import lax
from jax.experimental import pallas as pl
from jax.experimental.pallas import tpu as pltpu
from jax.experimental.pallas import tpu_sc as plsc

N = 48758
E = 780128
NP = 49152
EP = 786432
C = 2048
HID = 64
BLK = 512
GRID = NP // BLK
F32 = jnp.float32
I32 = jnp.int32

_mesh = plsc.VectorSubcoreMesh(
    core_axis_name="c", subcore_axis_name="s", num_cores=2, num_subcores=16
)
_sc_params = pltpu.CompilerParams(
    needs_layout_passes=False, use_tc_tiling_on_sc=False
)


def _k1_body(x_ref, w_ref, b_ref, h_ref, s_ref, q_ref):
    i = pl.program_id(0)
    h = jnp.dot(x_ref[...], w_ref[...], preferred_element_type=F32) + b_ref[...]
    h_ref[...] = h
    rid = i * BLK + lax.broadcasted_iota(I32, (BLK, 1), 0)
    hm = jnp.where(rid < N, h, 0.0)

    @pl.when(i == 0)
    def _():
        s_ref[...] = jnp.zeros_like(s_ref)
        q_ref[...] = jnp.zeros_like(q_ref)

    s_ref[0:1, :] += jnp.sum(hm, axis=0, keepdims=True)
    q_ref[0:1, :] += jnp.sum(hm * hm, axis=0, keepdims=True)


_k1 = pl.pallas_call(
    _k1_body,
    grid=(GRID,),
    in_specs=[
        pl.BlockSpec((BLK, 128), lambda i: (i, 0)),
        pl.BlockSpec((128, HID), lambda i: (0, 0)),
        pl.BlockSpec((1, HID), lambda i: (0, 0)),
    ],
    out_specs=[
        pl.BlockSpec((BLK, HID), lambda i: (i, 0)),
        pl.BlockSpec((8, HID), lambda i: (0, 0)),
        pl.BlockSpec((8, HID), lambda i: (0, 0)),
    ],
    out_shape=[
        jax.ShapeDtypeStruct((NP, HID), F32),
        jax.ShapeDtypeStruct((8, HID), F32),
        jax.ShapeDtypeStruct((8, HID), F32),
    ],
)


def _k2_body(h_ref, s_ref, q_ref, g_ref, bb_ref, wi_ref, bi_ref, wrel_ref,
             degp_ref, xf_ref, dis_ref):
    mean = s_ref[0:1, :] / N
    var = q_ref[0:1, :] / N - mean * mean
    rstd = lax.rsqrt(var + 1e-5)
    cc = jnp.maximum((h_ref[...] - mean) * rstd * g_ref[...] + bb_ref[...], 0.0)
    xin = jnp.dot(cc, wi_ref[...], preferred_element_type=F32) + bi_ref[...]
    deg = degp_ref[0] + degp_ref[1]
    dis = jnp.where(deg > 0, lax.rsqrt(deg), 0.0)
    dis_ref[...] = dis
    xp = xin * dis
    for r in range(4):
        xf_ref[r, :, 0:HID] = xp
        xf_ref[r, :, HID:2 * HID] = jnp.dot(
            xin, wrel_ref[r], preferred_element_type=F32)


_k2 = pl.pallas_call(
    _k2_body,
    grid=(GRID,),
    in_specs=[
        pl.BlockSpec((BLK, HID), lambda i: (i, 0)),
        pl.BlockSpec((8, HID), lambda i: (0, 0)),
        pl.BlockSpec((8, HID), lambda i: (0, 0)),
        pl.BlockSpec((1, HID), lambda i: (0, 0)),
        pl.BlockSpec((1, HID), lambda i: (0, 0)),
        pl.BlockSpec((HID, HID), lambda i: (0, 0)),
        pl.BlockSpec((1, HID), lambda i: (0, 0)),
        pl.BlockSpec((4, HID, HID), lambda i: (0, 0, 0)),
        pl.BlockSpec((2, BLK, 1), lambda i: (0, i, 0)),
    ],
    out_specs=[
        pl.BlockSpec((4, BLK, 2 * HID), lambda i: (0, i, 0)),
        pl.BlockSpec((BLK, 1), lambda i: (i, 0)),
    ],
    out_shape=[
        jax.ShapeDtypeStruct((4, NP, 2 * HID), F32),
        jax.ShapeDtypeStruct((NP, 1), F32),
    ],
)


def _k3_body(cb_ref, wo_ref, bo_ref, wi_ref, bi_ref, wrel_ref, dis_ref,
             xf_ref):
    c1 = jnp.dot(cb_ref[...], wo_ref[...], preferred_element_type=F32) + bo_ref[...]
    xin = jnp.dot(c1, wi_ref[...], preferred_element_type=F32) + bi_ref[...]
    xp = xin * dis_ref[...]
    for r in range(4):
        xf_ref[r, :, 0:HID] = xp
        xf_ref[r, :, HID:2 * HID] = jnp.dot(
            xin, wrel_ref[r], preferred_element_type=F32)


_k3 = pl.pallas_call(
    _k3_body,
    grid=(GRID,),
    in_specs=[
        pl.BlockSpec((BLK, HID), lambda i: (i, 0)),
        pl.BlockSpec((HID, HID), lambda i: (0, 0)),
        pl.BlockSpec((1, HID), lambda i: (0, 0)),
        pl.BlockSpec((HID, HID), lambda i: (0, 0)),
        pl.BlockSpec((1, HID), lambda i: (0, 0)),
        pl.BlockSpec((4, HID, HID), lambda i: (0, 0, 0)),
        pl.BlockSpec((BLK, 1), lambda i: (i, 0)),
    ],
    out_specs=[
        pl.BlockSpec((4, BLK, 2 * HID), lambda i: (0, i, 0)),
    ],
    out_shape=[
        jax.ShapeDtypeStruct((4, NP, 2 * HID), F32),
    ],
)


def _k5_body(rows_ref, wo_ref, bo_ref, out_ref):
    o = jnp.dot(rows_ref[...], wo_ref[...], preferred_element_type=F32) + bo_ref[...]
    out_ref[...] = 0.5 * o * (1.0 + lax.erf(o * (1.0 / math.sqrt(2.0))))


_k5 = pl.pallas_call(
    _k5_body,
    grid=(16,),
    in_specs=[
        pl.BlockSpec((BLK, HID), lambda i: (i, 0)),
        pl.BlockSpec((HID, HID), lambda i: (0, 0)),
        pl.BlockSpec((1, HID), lambda i: (0, 0)),
    ],
    out_specs=pl.BlockSpec((BLK, HID), lambda i: (i, 0)),
    out_shape=jax.ShapeDtypeStruct((8192, HID), F32),
)


@functools.partial(
    pl.kernel,
    mesh=_mesh,
    out_type=jax.ShapeDtypeStruct((2, NP), F32),
    scratch_types=[
        pltpu.VMEM((2048,), I32),
        pltpu.VMEM((128,), I32),
        pltpu.VMEM((128,), F32),
        pltpu.VMEM((3072,), F32),
        pltpu.VMEM_SHARED((NP,), F32),
    ],
    compiler_params=_sc_params,
)
def _deg_kernel(col_hbm, out_hbm, colstage, idxbuf, ones_v, zbuf, acc):
    cid = lax.axis_index("c")
    sid = lax.axis_index("s")
    wid = sid * 2 + cid

    def initz(i, _):
        zbuf[pl.ds(i * 16, 16)] = jnp.zeros((16,), F32)
        return 0

    lax.fori_loop(0, 192, initz, 0)
    for j in range(8):
        ones_v[pl.ds(j * 16, 16)] = jnp.ones((16,), F32)
    pltpu.sync_copy(zbuf, acc.at[pl.ds(sid * 3072, 3072)])
    plsc.subcore_barrier()

    base = wid * (EP // 32)

    def chunk_body(ch, _):
        pltpu.sync_copy(col_hbm.at[pl.ds(base + ch * 2048, 2048)], colstage)

        def batch_body(b, _2):
            for j in range(8):
                idxbuf[pl.ds(j * 16, 16)] = colstage[pl.ds(b * 128 + j * 16, 16)]
            pltpu.sync_copy(ones_v, acc.at[idxbuf], add=True)
            return 0

        lax.fori_loop(0, 16, batch_body, 0)
        return 0

    lax.fori_loop(0, (EP // 32) // 2048, chunk_body, 0)
    plsc.subcore_barrier()
    pltpu.sync_copy(
        acc.at[pl.ds(sid * 3072, 3072)], out_hbm.at[cid, pl.ds(sid * 3072, 3072)]
    )


_STAG = 2304


@functools.partial(
    pl.kernel,
    mesh=_mesh,
    out_type=jax.ShapeDtypeStruct((NP, HID), F32),
    scratch_types=[
        pltpu.VMEM((2048,), I32),
        pltpu.VMEM((2048,), I32),
        pltpu.VMEM((_STAG,), I32),
        pltpu.VMEM((_STAG,), I32),
        pltpu.VMEM((128,), I32),
        pltpu.VMEM((128,), I32),
        pltpu.VMEM((128, 2 * HID), F32),
        pltpu.VMEM((128, HID), F32),
        pltpu.VMEM((128, HID), F32),
        pltpu.VMEM((128, HID), F32),
        pltpu.VMEM((128, HID), F32),
        pltpu.VMEM((128, HID), F32),
        pltpu.VMEM((128,), F32),
        pltpu.VMEM_SHARED((C + 16, HID), F32),
        pltpu.VMEM_SHARED((C + 16, HID), F32),
        pltpu.VMEM_SHARED((C + 16, HID), F32),
        pltpu.SemaphoreType.DMA,
    ],
    compiler_params=_sc_params,
)
def _edge_kernel(row_hbm, ct_hbm, xf_hbm, dis_hbm, out_hbm,
                 rstage, ctstage, rls, xs, rlbuf, xbuf,
                 Frows, UA, US, UW, zbuf, ob, disloc,
                 accA, accS, accWS, semx):
    cid = lax.axis_index("c")
    sid = lax.axis_index("s")

    def initzb(e, _):
        zr = zbuf.at[e]
        for cch in range(4):
            zr[pl.ds(cch * 16, 16)] = jnp.zeros((16,), F32)
        return 0

    lax.fori_loop(0, 128, initzb, 0)

    def inits(i, _):
        fill = jnp.full((16,), 0, I32) + sid
        xs[pl.ds(i * 16, 16)] = fill
        rls[pl.ds(i * 16, 16)] = jnp.full((16,), C, I32)
        return 0

    lax.fori_loop(0, _STAG // 16, inits, 0)

    def process_batch(o):
        for j in range(8):
            rlbuf[pl.ds(j * 16, 16)] = rls[pl.ds(o + j * 16, 16)]
            xbuf[pl.ds(j * 16, 16)] = xs[pl.ds(o + j * 16, 16)]
        pltpu.async_copy(xf_hbm.at[xbuf], Frows, semx).wait()

        def ubody(e, _):
            fr = Frows.at[e]
            ua = UA.at[e]
            us = US.at[e]
            uw = UW.at[e]
            for cch in range(4):
                ua[pl.ds(cch * 16, 16)] = fr[pl.ds(cch * 16, 16)]
                v = fr[pl.ds(HID + cch * 16, 16)]
                p = jnp.exp(v)
                us[pl.ds(cch * 16, 16)] = p
                uw[pl.ds(cch * 16, 16)] = v * p
            return 0

        lax.fori_loop(0, 128, ubody, 0)
        pltpu.sync_copy(UA, accA.at[rlbuf], add=True)
        pltpu.sync_copy(US, accS.at[rlbuf], add=True)
        pltpu.sync_copy(UW, accWS.at[rlbuf], add=True)

    def pass_body(rid_pass, _pb):
        rid = rid_pass * 2 + cid
        base = rid * C

        r0 = sid * 129
        pltpu.sync_copy(zbuf, accA.at[pl.ds(r0, 128)])
        pltpu.sync_copy(zbuf, accS.at[pl.ds(r0, 128)])
        pltpu.sync_copy(zbuf, accWS.at[pl.ds(r0, 128)])
        pltpu.sync_copy(zbuf.at[pl.ds(0, 1)], accA.at[pl.ds(r0 + 128, 1)])
        pltpu.sync_copy(zbuf.at[pl.ds(0, 1)], accS.at[pl.ds(r0 + 128, 1)])
        pltpu.sync_copy(zbuf.at[pl.ds(0, 1)], accWS.at[pl.ds(r0 + 128, 1)])
        plsc.subcore_barrier()

        def chunk_body(ch, n_in):
            eb = sid * (EP // 16) + ch * 2048
            pltpu.sync_copy(row_hbm.at[pl.ds(eb, 2048)], rstage)
            pltpu.sync_copy(ct_hbm.at[pl.ds(eb, 2048)], ctstage)

            def grp(g, n):
                r16 = rstage[pl.ds(g * 16, 16)]
                ct16 = ctstage[pl.ds(g * 16, 16)]
                m = (r16 >= base) & (r16 < base + C)
                mi = jnp.where(m, 1, 0)
                cs_ = plsc.cumsum(mi)
                pos = n + cs_ - 1
                gi = lax.shift_right_logical(ct16, 16) * NP + (ct16 & 0xFFFF)
                plsc.store_scatter(rls, [pos], r16 - base, mask=m)
                plsc.store_scatter(xs, [pos], gi, mask=m)
                return n + cs_[15]

            n = lax.fori_loop(0, 128, grp, n_in)

            def dcond(st):
                return st[0] + 128 <= st[1]

            def dbody(st):
                nd, ns = st
                process_batch(nd)
                return (nd + 128, ns)

            nd, ns = lax.while_loop(dcond, dbody, (0, n))

            @pl.when(nd > 0)
            def _():
                for j in range(8):
                    rls[pl.ds(j * 16, 16)] = rls[pl.ds(nd + j * 16, 16)]
                    xs[pl.ds(j * 16, 16)] = xs[pl.ds(nd + j * 16, 16)]

            return ns - nd

        n_left = lax.fori_loop(0, (EP // 16) // 2048, chunk_body, 0)

        @pl.when(n_left > 0)
        def _():
            iota = jnp.arange(16, dtype=I32)
            dump = jnp.full((16,), C, I32)
            for j in range(8):
                plsc.store_scatter(rls, [n_left + j * 16 + iota], dump)
            process_batch(0)

        plsc.subcore_barrier()

        gbase = base + sid * 128
        pltpu.sync_copy(dis_hbm.at[pl.ds(gbase, 128)], disloc)
        pltpu.sync_copy(accA.at[pl.ds(sid * 128, 128)], UA)
        pltpu.sync_copy(accS.at[pl.ds(sid * 128, 128)], US)
        pltpu.sync_copy(accWS.at[pl.ds(sid * 128, 128)], UW)

        def pgroup(g, _):
            dv16 = disloc[pl.ds(g * 16, 16)]
            for k in range(16):
                i = g * 16 + k
                dv = dv16[k]
                ra = UA.at[i]
                rs = US.at[i]
                rw = UW.at[i]
                ro = ob.at[i]
                for cch in range(4):
                    a = ra[pl.ds(cch * 16, 16)]
                    s = rs[pl.ds(cch * 16, 16)]
                    w = rw[pl.ds(cch * 16, 16)]
                    msg = w / (s + 1e-16)
                    ro[pl.ds(cch * 16, 16)] = dv * a + 0.1 * jnp.maximum(msg, 0.0)
            return 0

        lax.fori_loop(0, 8, pgroup, 0)
        pltpu.sync_copy(ob, out_hbm.at[pl.ds(gbase, 128)])
        plsc.subcore_barrier()
        return 0

    lax.fori_loop(0, 12, pass_body, 0)


@functools.partial(
    pl.kernel,
    mesh=_mesh,
    out_type=jax.ShapeDtypeStruct((8192, HID), F32),
    scratch_types=[
        pltpu.VMEM((128,), I32),
        pltpu.VMEM((128, HID), F32),
        pltpu.SemaphoreType.DMA,
    ],
    compiler_params=_sc_params,
)
def _gather_kernel(src_hbm, idx_hbm, out_hbm, ib, rows, sem):
    cid = lax.axis_index("c")
    sid = lax.axis_index("s")
    wid = sid * 2 + cid
    for b in range(2):
        o = wid * 256 + b * 128
        pltpu.sync_copy(idx_hbm.at[pl.ds(o, 128)], ib)
        pltpu.async_copy(src_hbm.at[ib], rows, sem).wait()
        pltpu.sync_copy(rows, out_hbm.at[pl.ds(o, 128)])


def kernel(x, edge_index, idx, edge_type, edge_weight, proj_W, proj_b, bn_g,
           bn_b, Wi0, bi0, Wrel0, Wo0, bo0, Wi1, bi1, Wrel1, Wo1, bo1):
    del edge_weight
    x_p = jnp.pad(x, ((0, NP - N), (0, 0)))
    row = edge_index[0]
    col = edge_index[1]
    pad_e = EP - E
    ar = jnp.arange(pad_e, dtype=I32)
    row_p = jnp.concatenate([row, jnp.full((pad_e,), 1 << 20, I32)])
    col_p = jnp.concatenate([col, N + ar % (NP - N)])
    typ_p = jnp.concatenate([edge_type, jnp.zeros((pad_e,), I32)])
    ct_p = (typ_p << 16) | col_p

    degp = _deg_kernel(col_p)
    h, hsum, hsq = _k1(x_p, proj_W, proj_b.reshape(1, HID))
    xf0, dis = _k2(
        h, hsum, hsq, bn_g.reshape(1, HID), bn_b.reshape(1, HID), Wi0,
        bi0.reshape(1, HID), Wrel0, degp.reshape(2, NP, 1))
    dis_flat = dis.reshape(NP)
    comb0 = _edge_kernel(row_p, ct_p, xf0.reshape(4 * NP, 2 * HID), dis_flat)
    (xf1,) = _k3(comb0, Wo0, bo0.reshape(1, HID), Wi1,
                 bi1.reshape(1, HID), Wrel1, dis)
    comb1 = _edge_kernel(row_p, ct_p, xf1.reshape(4 * NP, 2 * HID), dis_flat)
    rows = _gather_kernel(comb1, idx)
    return _k5(rows, Wo1, bo1.reshape(1, HID))


# speedup vs baseline: 1.6385x; 1.6236x over previous
import functools
import math

import jax
import jax.numpy as jnp
from jax import lax
from jax.experimental import pallas as pl
from jax.experimental.pallas import tpu as pltpu
from jax.experimental.pallas import tpu_sc as plsc

N = 48758
E = 780128
NP = 49152
EP = 786432
C = 2048
HID = 64
BLK = 512
GRID = NP // BLK
F32 = jnp.float32
I32 = jnp.int32

_mesh = plsc.VectorSubcoreMesh(
    core_axis_name="c", subcore_axis_name="s", num_cores=2, num_subcores=16
)
_sc_params = pltpu.CompilerParams(
    needs_layout_passes=False, use_tc_tiling_on_sc=False
)


def _k1_body(x_ref, w_ref, b_ref, h_ref, s_ref, q_ref):
    i = pl.program_id(0)
    h = jnp.dot(x_ref[...], w_ref[...], preferred_element_type=F32) + b_ref[...]
    h_ref[...] = h
    rid = i * BLK + lax.broadcasted_iota(I32, (BLK, 1), 0)
    hm = jnp.where(rid < N, h, 0.0)

    @pl.when(i == 0)
    def _():
        s_ref[...] = jnp.zeros_like(s_ref)
        q_ref[...] = jnp.zeros_like(q_ref)

    s_ref[0:1, :] += jnp.sum(hm, axis=0, keepdims=True)
    q_ref[0:1, :] += jnp.sum(hm * hm, axis=0, keepdims=True)


_k1 = pl.pallas_call(
    _k1_body,
    grid=(GRID,),
    in_specs=[
        pl.BlockSpec((BLK, 128), lambda i: (i, 0)),
        pl.BlockSpec((128, HID), lambda i: (0, 0)),
        pl.BlockSpec((1, HID), lambda i: (0, 0)),
    ],
    out_specs=[
        pl.BlockSpec((BLK, HID), lambda i: (i, 0)),
        pl.BlockSpec((8, HID), lambda i: (0, 0)),
        pl.BlockSpec((8, HID), lambda i: (0, 0)),
    ],
    out_shape=[
        jax.ShapeDtypeStruct((NP, HID), F32),
        jax.ShapeDtypeStruct((8, HID), F32),
        jax.ShapeDtypeStruct((8, HID), F32),
    ],
)


def _k2_body(h_ref, s_ref, q_ref, g_ref, bb_ref, wi_ref, bi_ref, wrel_ref,
             degp_ref, xp_ref, xrel_ref, dis_ref):
    mean = s_ref[0:1, :] / N
    var = q_ref[0:1, :] / N - mean * mean
    rstd = lax.rsqrt(var + 1e-5)
    cc = jnp.maximum((h_ref[...] - mean) * rstd * g_ref[...] + bb_ref[...], 0.0)
    xin = jnp.dot(cc, wi_ref[...], preferred_element_type=F32) + bi_ref[...]
    deg = degp_ref[0] + degp_ref[1]
    dis = jnp.where(deg > 0, lax.rsqrt(deg), 0.0)
    dis_ref[...] = dis
    xp_ref[...] = xin * dis
    for r in range(4):
        xrel_ref[r] = jnp.dot(xin, wrel_ref[r], preferred_element_type=F32)


_k2 = pl.pallas_call(
    _k2_body,
    grid=(GRID,),
    in_specs=[
        pl.BlockSpec((BLK, HID), lambda i: (i, 0)),
        pl.BlockSpec((8, HID), lambda i: (0, 0)),
        pl.BlockSpec((8, HID), lambda i: (0, 0)),
        pl.BlockSpec((1, HID), lambda i: (0, 0)),
        pl.BlockSpec((1, HID), lambda i: (0, 0)),
        pl.BlockSpec((HID, HID), lambda i: (0, 0)),
        pl.BlockSpec((1, HID), lambda i: (0, 0)),
        pl.BlockSpec((4, HID, HID), lambda i: (0, 0, 0)),
        pl.BlockSpec((2, BLK, 1), lambda i: (0, i, 0)),
    ],
    out_specs=[
        pl.BlockSpec((BLK, HID), lambda i: (i, 0)),
        pl.BlockSpec((4, BLK, HID), lambda i: (0, i, 0)),
        pl.BlockSpec((BLK, 1), lambda i: (i, 0)),
    ],
    out_shape=[
        jax.ShapeDtypeStruct((NP, HID), F32),
        jax.ShapeDtypeStruct((4, NP, HID), F32),
        jax.ShapeDtypeStruct((NP, 1), F32),
    ],
)


def _k3_body(cb_ref, wo_ref, bo_ref, wi_ref, bi_ref, wrel_ref, dis_ref,
             xp_ref, xrel_ref):
    c1 = jnp.dot(cb_ref[...], wo_ref[...], preferred_element_type=F32) + bo_ref[...]
    xin = jnp.dot(c1, wi_ref[...], preferred_element_type=F32) + bi_ref[...]
    xp_ref[...] = xin * dis_ref[...]
    for r in range(4):
        xrel_ref[r] = jnp.dot(xin, wrel_ref[r], preferred_element_type=F32)


_k3 = pl.pallas_call(
    _k3_body,
    grid=(GRID,),
    in_specs=[
        pl.BlockSpec((BLK, HID), lambda i: (i, 0)),
        pl.BlockSpec((HID, HID), lambda i: (0, 0)),
        pl.BlockSpec((1, HID), lambda i: (0, 0)),
        pl.BlockSpec((HID, HID), lambda i: (0, 0)),
        pl.BlockSpec((1, HID), lambda i: (0, 0)),
        pl.BlockSpec((4, HID, HID), lambda i: (0, 0, 0)),
        pl.BlockSpec((BLK, 1), lambda i: (i, 0)),
    ],
    out_specs=[
        pl.BlockSpec((BLK, HID), lambda i: (i, 0)),
        pl.BlockSpec((4, BLK, HID), lambda i: (0, i, 0)),
    ],
    out_shape=[
        jax.ShapeDtypeStruct((NP, HID), F32),
        jax.ShapeDtypeStruct((4, NP, HID), F32),
    ],
)


def _k5_body(rows_ref, wo_ref, bo_ref, out_ref):
    o = jnp.dot(rows_ref[...], wo_ref[...], preferred_element_type=F32) + bo_ref[...]
    out_ref[...] = 0.5 * o * (1.0 + lax.erf(o * (1.0 / math.sqrt(2.0))))


_k5 = pl.pallas_call(
    _k5_body,
    grid=(16,),
    in_specs=[
        pl.BlockSpec((BLK, HID), lambda i: (i, 0)),
        pl.BlockSpec((HID, HID), lambda i: (0, 0)),
        pl.BlockSpec((1, HID), lambda i: (0, 0)),
    ],
    out_specs=pl.BlockSpec((BLK, HID), lambda i: (i, 0)),
    out_shape=jax.ShapeDtypeStruct((8192, HID), F32),
)


@functools.partial(
    pl.kernel,
    mesh=_mesh,
    out_type=jax.ShapeDtypeStruct((2, NP), F32),
    scratch_types=[
        pltpu.VMEM((2048,), I32),
        pltpu.VMEM((128,), I32),
        pltpu.VMEM((128,), F32),
        pltpu.VMEM((3072,), F32),
        pltpu.VMEM_SHARED((NP,), F32),
    ],
    compiler_params=_sc_params,
)
def _deg_kernel(col_hbm, out_hbm, colstage, idxbuf, ones_v, zbuf, acc):
    cid = lax.axis_index("c")
    sid = lax.axis_index("s")
    wid = sid * 2 + cid

    def initz(i, _):
        zbuf[pl.ds(i * 16, 16)] = jnp.zeros((16,), F32)
        return 0

    lax.fori_loop(0, 192, initz, 0)
    for j in range(8):
        ones_v[pl.ds(j * 16, 16)] = jnp.ones((16,), F32)
    pltpu.sync_copy(zbuf, acc.at[pl.ds(sid * 3072, 3072)])
    plsc.subcore_barrier()

    base = wid * (EP // 32)

    def chunk_body(ch, _):
        pltpu.sync_copy(col_hbm.at[pl.ds(base + ch * 2048, 2048)], colstage)

        def batch_body(b, _2):
            for j in range(8):
                idxbuf[pl.ds(j * 16, 16)] = colstage[pl.ds(b * 128 + j * 16, 16)]
            pltpu.sync_copy(ones_v, acc.at[idxbuf], add=True)
            return 0

        lax.fori_loop(0, 16, batch_body, 0)
        return 0

    lax.fori_loop(0, (EP // 32) // 2048, chunk_body, 0)
    plsc.subcore_barrier()
    pltpu.sync_copy(
        acc.at[pl.ds(sid * 3072, 3072)], out_hbm.at[cid, pl.ds(sid * 3072, 3072)]
    )


_STAG = 2304


@functools.partial(
    pl.kernel,
    mesh=_mesh,
    out_type=jax.ShapeDtypeStruct((NP, HID), F32),
    scratch_types=[
        pltpu.VMEM((2048,), I32),
        pltpu.VMEM((2048,), I32),
        pltpu.VMEM((_STAG,), I32),
        pltpu.VMEM((_STAG,), I32),
        pltpu.VMEM((_STAG,), I32),
        pltpu.VMEM((64,), I32),
        pltpu.VMEM((64,), I32),
        pltpu.VMEM((64,), I32),
        pltpu.VMEM((64,), I32),
        pltpu.VMEM((64,), I32),
        pltpu.VMEM((64,), I32),
        pltpu.VMEM((64, HID), F32),
        pltpu.VMEM((64, HID), F32),
        pltpu.VMEM((64, HID), F32),
        pltpu.VMEM((64, HID), F32),
        pltpu.VMEM((64, HID), F32),
        pltpu.VMEM((64, HID), F32),
        pltpu.VMEM((64, HID), F32),
        pltpu.VMEM((64, HID), F32),
        pltpu.VMEM((128, HID), F32),
        pltpu.VMEM((128, HID), F32),
        pltpu.VMEM((128,), F32),
        pltpu.VMEM((128, HID), F32),
        pltpu.VMEM((128, HID), F32),
        pltpu.VMEM((128, HID), F32),
        pltpu.VMEM_SHARED((C + 16, HID), F32),
        pltpu.VMEM_SHARED((C + 16, HID), F32),
        pltpu.VMEM_SHARED((C + 16, HID), F32),
        pltpu.SemaphoreType.DMA,
        pltpu.SemaphoreType.DMA,
        pltpu.SemaphoreType.DMA,
        pltpu.SemaphoreType.DMA,
    ],
    compiler_params=_sc_params,
)
def _edge_kernel(row_hbm, ct_hbm, xp_hbm, xrel_hbm, dis_hbm, out_hbm,
                 rstage, ctstage, rls, cs, xs,
                 rlbufA, rlbufB, cbufA, cbufB, xbufA, xbufB,
                 XrowsA, XrowsB, RrowsA, RrowsB, USA, USB, UWA, UWB,
                 zbuf, ob, disloc, pA, pS, pW,
                 accA, accS, accWS, semXA, semXB, semRA, semRB):
    cid = lax.axis_index("c")
    sid = lax.axis_index("s")

    def initzb(e, _):
        zr = zbuf.at[e]
        for cch in range(4):
            zr[pl.ds(cch * 16, 16)] = jnp.zeros((16,), F32)
        return 0

    lax.fori_loop(0, 128, initzb, 0)

    def inits(i, _):
        fill = jnp.full((16,), 0, I32) + sid
        cs[pl.ds(i * 16, 16)] = fill
        xs[pl.ds(i * 16, 16)] = fill
        rls[pl.ds(i * 16, 16)] = jnp.full((16,), C, I32)
        return 0

    lax.fori_loop(0, _STAG // 16, inits, 0)

    def process_batch(o):
        for j in range(4):
            rlbufA[pl.ds(j * 16, 16)] = rls[pl.ds(o + j * 16, 16)]
            rlbufB[pl.ds(j * 16, 16)] = rls[pl.ds(o + 64 + j * 16, 16)]
            cbufA[pl.ds(j * 16, 16)] = cs[pl.ds(o + j * 16, 16)]
            cbufB[pl.ds(j * 16, 16)] = cs[pl.ds(o + 64 + j * 16, 16)]
            xbufA[pl.ds(j * 16, 16)] = xs[pl.ds(o + j * 16, 16)]
            xbufB[pl.ds(j * 16, 16)] = xs[pl.ds(o + 64 + j * 16, 16)]
        dxa = pltpu.async_copy(xp_hbm.at[cbufA], XrowsA, semXA)
        dxb = pltpu.async_copy(xp_hbm.at[cbufB], XrowsB, semXB)
        dra = pltpu.async_copy(xrel_hbm.at[xbufA], RrowsA, semRA)
        drb = pltpu.async_copy(xrel_hbm.at[xbufB], RrowsB, semRB)
        dxa.wait()
        dxb.wait()
        dra.wait()
        drb.wait()

        def ubodyA(e, _):
            rr = RrowsA.at[e]
            us = USA.at[e]
            uw = UWA.at[e]
            for cch in range(4):
                v = rr[pl.ds(cch * 16, 16)]
                p = jnp.exp(v)
                us[pl.ds(cch * 16, 16)] = p
                uw[pl.ds(cch * 16, 16)] = v * p
            return 0

        def ubodyB(e, _):
            rr = RrowsB.at[e]
            us = USB.at[e]
            uw = UWB.at[e]
            for cch in range(4):
                v = rr[pl.ds(cch * 16, 16)]
                p = jnp.exp(v)
                us[pl.ds(cch * 16, 16)] = p
                uw[pl.ds(cch * 16, 16)] = v * p
            return 0

        lax.fori_loop(0, 64, ubodyA, 0)
        lax.fori_loop(0, 64, ubodyB, 0)
        pltpu.sync_copy(XrowsA, accA.at[rlbufA], add=True)
        pltpu.sync_copy(XrowsB, accA.at[rlbufB], add=True)
        pltpu.sync_copy(USA, accS.at[rlbufA], add=True)
        pltpu.sync_copy(USB, accS.at[rlbufB], add=True)
        pltpu.sync_copy(UWA, accWS.at[rlbufA], add=True)
        pltpu.sync_copy(UWB, accWS.at[rlbufB], add=True)

    def pass_body(rid_pass, _pb):
        rid = rid_pass * 2 + cid
        base = rid * C

        r0 = sid * 129
        pltpu.sync_copy(zbuf, accA.at[pl.ds(r0, 128)])
        pltpu.sync_copy(zbuf, accS.at[pl.ds(r0, 128)])
        pltpu.sync_copy(zbuf, accWS.at[pl.ds(r0, 128)])
        pltpu.sync_copy(zbuf.at[pl.ds(0, 1)], accA.at[pl.ds(r0 + 128, 1)])
        pltpu.sync_copy(zbuf.at[pl.ds(0, 1)], accS.at[pl.ds(r0 + 128, 1)])
        pltpu.sync_copy(zbuf.at[pl.ds(0, 1)], accWS.at[pl.ds(r0 + 128, 1)])
        plsc.subcore_barrier()

        def chunk_body(ch, n_in):
            eb = sid * (EP // 16) + ch * 2048
            pltpu.sync_copy(row_hbm.at[pl.ds(eb, 2048)], rstage)
            pltpu.sync_copy(ct_hbm.at[pl.ds(eb, 2048)], ctstage)

            def grp(g, n):
                r16 = rstage[pl.ds(g * 16, 16)]
                ct16 = ctstage[pl.ds(g * 16, 16)]
                m = (r16 >= base) & (r16 < base + C)
                mi = jnp.where(m, 1, 0)
                cs_ = plsc.cumsum(mi)
                pos = n + cs_ - 1
                c16 = ct16 & 0xFFFF
                gi = lax.shift_right_logical(ct16, 16) * NP + c16
                plsc.store_scatter(rls, [pos], r16 - base, mask=m)
                plsc.store_scatter(cs, [pos], c16, mask=m)
                plsc.store_scatter(xs, [pos], gi, mask=m)
                return n + cs_[15]

            n = lax.fori_loop(0, 128, grp, n_in)

            def dcond(st):
                return st[0] + 128 <= st[1]

            def dbody(st):
                nd, ns = st
                process_batch(nd)
                return (nd + 128, ns)

            nd, ns = lax.while_loop(dcond, dbody, (0, n))

            @pl.when(nd > 0)
            def _():
                for j in range(8):
                    rls[pl.ds(j * 16, 16)] = rls[pl.ds(nd + j * 16, 16)]
                    cs[pl.ds(j * 16, 16)] = cs[pl.ds(nd + j * 16, 16)]
                    xs[pl.ds(j * 16, 16)] = xs[pl.ds(nd + j * 16, 16)]

            return ns - nd

        n_left = lax.fori_loop(0, (EP // 16) // 2048, chunk_body, 0)

        @pl.when(n_left > 0)
        def _():
            iota = jnp.arange(16, dtype=I32)
            dump = jnp.full((16,), C, I32)
            for j in range(8):
                plsc.store_scatter(rls, [n_left + j * 16 + iota], dump)
            process_batch(0)

        plsc.subcore_barrier()

        gbase = base + sid * 128
        pltpu.sync_copy(dis_hbm.at[pl.ds(gbase, 128)], disloc)
        pltpu.sync_copy(accA.at[pl.ds(sid * 128, 128)], pA)
        pltpu.sync_copy(accS.at[pl.ds(sid * 128, 128)], pS)
        pltpu.sync_copy(accWS.at[pl.ds(sid * 128, 128)], pW)

        def pgroup(g, _):
            dv16 = disloc[pl.ds(g * 16, 16)]
            for k in range(16):
                i = g * 16 + k
                dv = dv16[k]
                ra = pA.at[i]
                rs = pS.at[i]
                rw = pW.at[i]
                ro = ob.at[i]
                for cch in range(4):
                    a = ra[pl.ds(cch * 16, 16)]
                    s = rs[pl.ds(cch * 16, 16)]
                    w = rw[pl.ds(cch * 16, 16)]
                    msg = w / (s + 1e-16)
                    ro[pl.ds(cch * 16, 16)] = dv * a + 0.1 * jnp.maximum(msg, 0.0)
            return 0

        lax.fori_loop(0, 8, pgroup, 0)
        pltpu.sync_copy(ob, out_hbm.at[pl.ds(gbase, 128)])
        plsc.subcore_barrier()
        return 0

    lax.fori_loop(0, 12, pass_body, 0)


@functools.partial(
    pl.kernel,
    mesh=_mesh,
    out_type=jax.ShapeDtypeStruct((8192, HID), F32),
    scratch_types=[
        pltpu.VMEM((128,), I32),
        pltpu.VMEM((128, HID), F32),
        pltpu.SemaphoreType.DMA,
    ],
    compiler_params=_sc_params,
)
def _gather_kernel(src_hbm, idx_hbm, out_hbm, ib, rows, sem):
    cid = lax.axis_index("c")
    sid = lax.axis_index("s")
    wid = sid * 2 + cid
    for b in range(2):
        o = wid * 256 + b * 128
        pltpu.sync_copy(idx_hbm.at[pl.ds(o, 128)], ib)
        pltpu.async_copy(src_hbm.at[ib], rows, sem).wait()
        pltpu.sync_copy(rows, out_hbm.at[pl.ds(o, 128)])


def kernel(x, edge_index, idx, edge_type, edge_weight, proj_W, proj_b, bn_g,
           bn_b, Wi0, bi0, Wrel0, Wo0, bo0, Wi1, bi1, Wrel1, Wo1, bo1):
    del edge_weight
    x_p = jnp.pad(x, ((0, NP - N), (0, 0)))
    row = edge_index[0]
    col = edge_index[1]
    pad_e = EP - E
    ar = jnp.arange(pad_e, dtype=I32)
    row_p = jnp.concatenate([row, jnp.full((pad_e,), 1 << 20, I32)])
    col_p = jnp.concatenate([col, N + ar % (NP - N)])
    typ_p = jnp.concatenate([edge_type, jnp.zeros((pad_e,), I32)])
    ct_p = (typ_p << 16) | col_p

    degp = _deg_kernel(col_p)
    h, hsum, hsq = _k1(x_p, proj_W, proj_b.reshape(1, HID))
    xp0, xrel0, dis = _k2(
        h, hsum, hsq, bn_g.reshape(1, HID), bn_b.reshape(1, HID), Wi0,
        bi0.reshape(1, HID), Wrel0, degp.reshape(2, NP, 1))
    dis_flat = dis.reshape(NP)
    comb0 = _edge_kernel(row_p, ct_p, xp0, xrel0.reshape(4 * NP, HID),
                         dis_flat)
    xp1, xrel1 = _k3(comb0, Wo0, bo0.reshape(1, HID), Wi1,
                     bi1.reshape(1, HID), Wrel1, dis)
    comb1 = _edge_kernel(row_p, ct_p, xp1, xrel1.reshape(4 * NP, HID),
                         dis_flat)
    rows = _gather_kernel(comb1, idx)
    return _k5(rows, Wo1, bo1.reshape(1, HID))
